# Initial kernel scaffold; baseline (speedup 1.0000x reference)
import jax
import jax.numpy as jnp
from jax.experimental import pallas as pl


def kernel(x, edge_index, edge_attr, W_ne1, b_ne1, W_ne2, b_ne2, Wc1, bc1, Wc2, bc2, Wc3, bc3, Wp1, bp1, Wp2, bp2, Wf1, bf1, Wf2, bf2, Ws1, bs1, Ws2, bs2):
    raise NotImplementedError("write your pallas kernel here")


# speedup vs baseline: 8.2722x; 8.2722x over previous
import functools

import jax
import jax.numpy as jnp
from jax import lax
from jax.experimental import pallas as pl
from jax.experimental.pallas import tpu as pltpu
from jax.experimental.pallas import tpu_sc as plsc

N = 10000
E = 320000
D = 256
HD = 128
NSUB = 16
SR = 624
TAIL = N - NSUB * SR
K = 80

BM = 400
NB = N // BM


def _sc_mesh():
    return plsc.VectorSubcoreMesh(core_axis_name="c", subcore_axis_name="s")


def _stripe_copy(src_ref, src_base, dst_ref, dst_base, s):
    o1 = pl.multiple_of(src_base + s * SR, 8)
    o2 = pl.multiple_of(dst_base + s * SR, 8)
    pltpu.sync_copy(src_ref.at[pl.ds(o1, SR)], dst_ref.at[pl.ds(o2, SR)])

    @pl.when(s == NSUB - 1)
    def _():
        t1 = pl.multiple_of(src_base + NSUB * SR, 8)
        t2 = pl.multiple_of(dst_base + NSUB * SR, 8)
        pltpu.sync_copy(src_ref.at[pl.ds(t1, TAIL)],
                        dst_ref.at[pl.ds(t2, TAIL)])


def _sc_degree(dst):
    z = jnp.zeros((N, HD), jnp.float32)
    ones = jnp.ones((K, HD), jnp.float32)
    ept = (E // 2) // NSUB

    @functools.partial(
        pl.kernel,
        mesh=_sc_mesh(),
        out_type=jax.ShapeDtypeStruct((2 * N, HD), jnp.float32),
        scratch_types=[
            pltpu.VMEM((K,), jnp.int32),
            pltpu.VMEM((K, HD), jnp.float32),
            pltpu.VMEM_SHARED((N, HD), jnp.float32),
        ],
    )
    def k(dst_hbm, z_hbm, ones_hbm, out_hbm, dst_v, ones_v, deg_sh):
        c = lax.axis_index("c")
        s = lax.axis_index("s")
        _stripe_copy(z_hbm, 0, deg_sh, 0, s)
        pltpu.sync_copy(ones_hbm, ones_v)
        plsc.subcore_barrier()
        base = c * (E // 2) + s * ept

        def body(j, carry):
            pltpu.sync_copy(dst_hbm.at[pl.ds(base + j * K, K)], dst_v)
            pltpu.sync_copy(ones_v, deg_sh.at[dst_v], add=True)
            return carry

        lax.fori_loop(0, ept // K, body, 0)
        plsc.subcore_barrier()
        _stripe_copy(deg_sh, 0, out_hbm, c * N, s)

    return k(dst, z, ones)


def _sc_segsum(g2, src2, dst):
    z = jnp.zeros((N, HD), jnp.float32)
    ept = E // NSUB

    @functools.partial(
        pl.kernel,
        mesh=_sc_mesh(),
        out_type=jax.ShapeDtypeStruct((2 * N, HD), jnp.float32),
        scratch_types=[
            pltpu.VMEM((K,), jnp.int32),
            pltpu.VMEM((K,), jnp.int32),
            pltpu.VMEM((K, HD), jnp.float32),
            pltpu.VMEM_SHARED((N, HD), jnp.float32),
            pltpu.SemaphoreType.DMA,
        ],
    )
    def k(g2_hbm, src2_hbm, dst_hbm, z_hbm, out_hbm,
          src_v, dst_v, rows_v, acc_sh, sem):
        c = lax.axis_index("c")
        s = lax.axis_index("s")
        _stripe_copy(z_hbm, 0, acc_sh, 0, s)
        plsc.subcore_barrier()
        ebase = s * ept
        sbase = c * E + ebase

        def body(j, carry):
            pltpu.sync_copy(src2_hbm.at[pl.ds(sbase + j * K, K)], src_v)
            pltpu.sync_copy(dst_hbm.at[pl.ds(ebase + j * K, K)], dst_v)
            pltpu.async_copy(g2_hbm.at[src_v], rows_v, sem).wait()
            pltpu.sync_copy(rows_v, acc_sh.at[dst_v], add=True)
            return carry

        lax.fori_loop(0, ept // K, body, 0)
        plsc.subcore_barrier()
        _stripe_copy(acc_sh, 0, out_hbm, c * N, s)

    return k(g2, src2, dst, z)


def _enc(x, W1, b1, W2, b2):
    def body(x_ref, w1_ref, b1_ref, w2_ref, b2_ref, o_ref):
        t = jnp.dot(x_ref[...], w1_ref[...], preferred_element_type=jnp.float32)
        t = jnp.maximum(t + b1_ref[...], 0.0)
        o_ref[...] = (jnp.dot(t, w2_ref[...], preferred_element_type=jnp.float32)
                      + b2_ref[...])

    return pl.pallas_call(
        body,
        grid=(NB,),
        in_specs=[
            pl.BlockSpec((BM, 128), lambda i: (i, 0)),
            pl.BlockSpec((128, D), lambda i: (0, 0)),
            pl.BlockSpec((1, D), lambda i: (0, 0)),
            pl.BlockSpec((D, D), lambda i: (0, 0)),
            pl.BlockSpec((1, D), lambda i: (0, 0)),
        ],
        out_specs=pl.BlockSpec((BM, D), lambda i: (i, 0)),
        out_shape=jax.ShapeDtypeStruct((N, D), jnp.float32),
    )(x, W1, b1.reshape(1, D), W2, b2.reshape(1, D))


def _gmm(h, W, deg2):
    def body(h_ref, w_ref, da_ref, db_ref, o_ref):
        dinv = lax.rsqrt(da_ref[...] + db_ref[...] + 1.0)
        g = dinv * jnp.dot(h_ref[...], w_ref[...],
                           preferred_element_type=jnp.float32)
        o_ref[0] = g[:, :HD]
        o_ref[1] = g[:, HD:]

    return pl.pallas_call(
        body,
        grid=(NB,),
        in_specs=[
            pl.BlockSpec((BM, D), lambda i: (i, 0)),
            pl.BlockSpec((D, D), lambda i: (0, 0)),
            pl.BlockSpec((BM, 1), lambda i: (i, 0)),
            pl.BlockSpec((BM, 1), lambda i: (NB + i, 0)),
        ],
        out_specs=pl.BlockSpec((2, BM, HD), lambda i: (0, i, 0)),
        out_shape=jax.ShapeDtypeStruct((2, N, HD), jnp.float32),
    )(h, W, deg2, deg2)


def _post(acc2, g2, deg2, b):
    def body(aL, aR, gL, gR, da, db, b_ref, o_ref):
        dinv = lax.rsqrt(da[...] + db[...] + 1.0)
        o_ref[:, :HD] = jnp.maximum(
            dinv * (aL[...] + gL[...]) + b_ref[:, :HD], 0.0)
        o_ref[:, HD:] = jnp.maximum(
            dinv * (aR[...] + gR[...]) + b_ref[:, HD:], 0.0)

    return pl.pallas_call(
        body,
        grid=(NB,),
        in_specs=[
            pl.BlockSpec((BM, HD), lambda i: (i, 0)),
            pl.BlockSpec((BM, HD), lambda i: (NB + i, 0)),
            pl.BlockSpec((BM, HD), lambda i: (i, 0)),
            pl.BlockSpec((BM, HD), lambda i: (NB + i, 0)),
            pl.BlockSpec((BM, 1), lambda i: (i, 0)),
            pl.BlockSpec((BM, 1), lambda i: (NB + i, 0)),
            pl.BlockSpec((1, D), lambda i: (0, 0)),
        ],
        out_specs=pl.BlockSpec((BM, D), lambda i: (i, 0)),
        out_shape=jax.ShapeDtypeStruct((N, D), jnp.float32),
    )(acc2, acc2, g2, g2, deg2, deg2, b.reshape(1, D))


def _head(h, idx2, ea0, Wp1, bp1, Wp2, bp2, Wf1, bf1, Wf2, bf2,
          Ws1, bs1, Ws2, bs2):
    def body(idx_ref, hs_ref, hd_ref, ea_ref,
             wp1, bp1r, wp2, bp2r, wf1, bf1r, wf2, bf2r,
             ws1, bs1r, ws2, bs2r, c_ref, s_ref):
        rs = idx_ref[0] % 8
        rd = idx_ref[1] % 8
        hs = hs_ref[pl.ds(rs, 1), :]
        hd = hd_ref[pl.ds(rd, 1), :]
        dot = functools.partial(jnp.dot, preferred_element_type=jnp.float32)
        e = dot(hs, wp1[:D, :]) + dot(hd, wp1[D:, :]) + bp1r[...]
        e = jnp.maximum(e, 0.0)
        e = dot(e, wp2[...]) + bp2r[...]
        f = dot(e, wf1[:HD, :]) + dot(ea_ref[...], wf1[HD:, :]) + bf1r[...]
        f = jnp.maximum(f, 0.0)
        f = dot(f, wf2[...]) + bf2r[...]
        c_ref[...] = f
        s = jnp.maximum(dot(f, ws1[...]) + bs1r[...], 0.0)
        z = dot(s, ws2[...]) + bs2r[...]
        s_ref[...] = 1.0 / (1.0 + jnp.exp(-z))

    grid_spec = pltpu.PrefetchScalarGridSpec(
        num_scalar_prefetch=1,
        grid=(1,),
        in_specs=[
            pl.BlockSpec((8, D), lambda i, idx: (idx[0] // 8, 0)),
            pl.BlockSpec((8, D), lambda i, idx: (idx[1] // 8, 0)),
            pl.BlockSpec((1, 16), lambda i, idx: (0, 0)),
            pl.BlockSpec((2 * D, D), lambda i, idx: (0, 0)),
            pl.BlockSpec((1, D), lambda i, idx: (0, 0)),
            pl.BlockSpec((D, HD), lambda i, idx: (0, 0)),
            pl.BlockSpec((1, HD), lambda i, idx: (0, 0)),
            pl.BlockSpec((HD + 16, D), lambda i, idx: (0, 0)),
            pl.BlockSpec((1, D), lambda i, idx: (0, 0)),
            pl.BlockSpec((D, HD), lambda i, idx: (0, 0)),
            pl.BlockSpec((1, HD), lambda i, idx: (0, 0)),
            pl.BlockSpec((HD, D), lambda i, idx: (0, 0)),
            pl.BlockSpec((1, D), lambda i, idx: (0, 0)),
            pl.BlockSpec((D, 1), lambda i, idx: (0, 0)),
            pl.BlockSpec((1, 1), lambda i, idx: (0, 0)),
        ],
        out_specs=[
            pl.BlockSpec((1, HD), lambda i, idx: (0, 0)),
            pl.BlockSpec((1, 1), lambda i, idx: (0, 0)),
        ],
    )
    return pl.pallas_call(
        body,
        grid_spec=grid_spec,
        out_shape=[
            jax.ShapeDtypeStruct((1, HD), jnp.float32),
            jax.ShapeDtypeStruct((1, 1), jnp.float32),
        ],
    )(idx2, h, h, ea0,
      Wp1, bp1.reshape(1, D), Wp2, bp2.reshape(1, HD),
      Wf1, bf1.reshape(1, D), Wf2, bf2.reshape(1, HD),
      Ws1, bs1.reshape(1, D), Ws2, bs2.reshape(1, 1))


def kernel(x, edge_index, edge_attr,
           W_ne1, b_ne1, W_ne2, b_ne2,
           Wc1, bc1, Wc2, bc2, Wc3, bc3,
           Wp1, bp1, Wp2, bp2,
           Wf1, bf1, Wf2, bf2,
           Ws1, bs1, Ws2, bs2):
    ei = edge_index.astype(jnp.int32)
    src, dst = ei[0], ei[1]
    src2 = jnp.concatenate([src, src + N])
    idx2 = ei[:, 0]
    ea0 = edge_attr[0:1]

    deg2 = _sc_degree(dst)[:, :1]
    h = _enc(x, W_ne1, b_ne1, W_ne2, b_ne2)
    for (W, b) in ((Wc1, bc1), (Wc2, bc2), (Wc3, bc3)):
        g2 = _gmm(h, W, deg2).reshape(2 * N, HD)
        acc2 = _sc_segsum(g2, src2, dst)
        h = _post(acc2, g2, deg2, b)

    center, score = _head(h, idx2, ea0, Wp1, bp1, Wp2, bp2,
                          Wf1, bf1, Wf2, bf2, Ws1, bs1, Ws2, bs2)
    return center, score.reshape(1)


# speedup vs baseline: 14.7170x; 1.7791x over previous
import functools

import jax
import jax.numpy as jnp
from jax import lax
from jax.experimental import pallas as pl
from jax.experimental.pallas import tpu as pltpu
from jax.experimental.pallas import tpu_sc as plsc

N = 10000
E = 320000
D = 256
HD = 128
NSUB = 16
SR = 624
TAIL = N - NSUB * SR
K = 80

BM = 400
NB = N // BM


def _sc_mesh():
    return plsc.VectorSubcoreMesh(core_axis_name="c", subcore_axis_name="s")


def _stripe_copy(src_ref, src_base, dst_ref, dst_base, s):
    o1 = pl.multiple_of(src_base + s * SR, 8)
    o2 = pl.multiple_of(dst_base + s * SR, 8)
    pltpu.sync_copy(src_ref.at[pl.ds(o1, SR)], dst_ref.at[pl.ds(o2, SR)])

    @pl.when(s == NSUB - 1)
    def _():
        t1 = pl.multiple_of(src_base + NSUB * SR, 8)
        t2 = pl.multiple_of(dst_base + NSUB * SR, 8)
        pltpu.sync_copy(src_ref.at[pl.ds(t1, TAIL)],
                        dst_ref.at[pl.ds(t2, TAIL)])


def _sc_degree(dst):
    z = jnp.zeros((N, HD), jnp.float32)
    ones = jnp.ones((K, HD), jnp.float32)
    ept = (E // 2) // NSUB

    @functools.partial(
        pl.kernel,
        mesh=_sc_mesh(),
        out_type=jax.ShapeDtypeStruct((2 * N, HD), jnp.float32),
        scratch_types=[
            pltpu.VMEM((K,), jnp.int32),
            pltpu.VMEM((K, HD), jnp.float32),
            pltpu.VMEM_SHARED((N, HD), jnp.float32),
        ],
    )
    def k(dst_hbm, z_hbm, ones_hbm, out_hbm, dst_v, ones_v, deg_sh):
        c = lax.axis_index("c")
        s = lax.axis_index("s")
        _stripe_copy(z_hbm, 0, deg_sh, 0, s)
        pltpu.sync_copy(ones_hbm, ones_v)
        plsc.subcore_barrier()
        base = c * (E // 2) + s * ept

        def body(j, carry):
            pltpu.sync_copy(dst_hbm.at[pl.ds(base + j * K, K)], dst_v)
            pltpu.sync_copy(ones_v, deg_sh.at[dst_v], add=True)
            return carry

        lax.fori_loop(0, ept // K, body, 0)
        plsc.subcore_barrier()
        _stripe_copy(deg_sh, 0, out_hbm, c * N, s)

    return k(dst, z, ones)


CPT = E // K // NSUB
G = 10
NG = CPT // G


def _sc_segsum(g2, src3, dst3):
    z = jnp.zeros((N, HD), jnp.float32)

    @functools.partial(
        pl.kernel,
        mesh=_sc_mesh(),
        out_type=jax.ShapeDtypeStruct((2 * N, HD), jnp.float32),
        scratch_types=[
            pltpu.VMEM((2, G, K), jnp.int32),
            pltpu.VMEM((2, G, K), jnp.int32),
            pltpu.VMEM((2, K, HD), jnp.float32),
            pltpu.VMEM_SHARED((N, HD), jnp.float32),
            pltpu.SemaphoreType.DMA,
            pltpu.SemaphoreType.DMA,
        ],
    )
    def k(g2_hbm, src3_hbm, dst3_hbm, z_hbm, out_hbm,
          srcg, dstg, rows_v, acc_sh, sem_g, sem_i):
        c = lax.axis_index("c")
        s = lax.axis_index("s")
        w = c * NSUB + s
        pltpu.sync_copy(src3_hbm.at[w * NG], srcg.at[0])
        pltpu.sync_copy(dst3_hbm.at[s * NG], dstg.at[0])
        _stripe_copy(z_hbm, 0, acc_sh, 0, s)
        plsc.subcore_barrier()

        pltpu.async_copy(g2_hbm.at[srcg.at[0].at[0]], rows_v.at[0], sem_g)

        def body(gi, carry):
            gp = gi % 2

            @pl.when(gi + 1 < NG)
            def _():
                pltpu.async_copy(src3_hbm.at[w * NG + gi + 1],
                                 srcg.at[(gi + 1) % 2], sem_i)
                pltpu.async_copy(dst3_hbm.at[s * NG + gi + 1],
                                 dstg.at[(gi + 1) % 2], sem_i)

            for j in range(G):
                pltpu.make_async_copy(g2_hbm.at[pl.ds(0, K)],
                                      rows_v.at[j % 2], sem_g).wait()
                if j < G - 1:
                    pltpu.async_copy(g2_hbm.at[srcg.at[gp].at[j + 1]],
                                     rows_v.at[(j + 1) % 2], sem_g)
                else:
                    @pl.when(gi + 1 < NG)
                    def _():
                        pltpu.make_async_copy(src3_hbm.at[0],
                                              srcg.at[0], sem_i).wait()
                        pltpu.make_async_copy(dst3_hbm.at[0],
                                              dstg.at[0], sem_i).wait()
                        pltpu.async_copy(
                            g2_hbm.at[srcg.at[(gi + 1) % 2].at[0]],
                            rows_v.at[(j + 1) % 2], sem_g)
                pltpu.sync_copy(rows_v.at[j % 2],
                                acc_sh.at[dstg.at[gp].at[j]], add=True)
            return carry

        lax.fori_loop(0, NG, body, 0)
        plsc.subcore_barrier()
        _stripe_copy(acc_sh, 0, out_hbm, c * N, s)

    return k(g2, src3, dst3, z)


def _enc(x, W1, b1, W2, b2):
    def body(x_ref, w1_ref, b1_ref, w2_ref, b2_ref, o_ref):
        t = jnp.dot(x_ref[...], w1_ref[...], preferred_element_type=jnp.float32)
        t = jnp.maximum(t + b1_ref[...], 0.0)
        o_ref[...] = (jnp.dot(t, w2_ref[...], preferred_element_type=jnp.float32)
                      + b2_ref[...])

    return pl.pallas_call(
        body,
        grid=(NB,),
        in_specs=[
            pl.BlockSpec((BM, 128), lambda i: (i, 0)),
            pl.BlockSpec((128, D), lambda i: (0, 0)),
            pl.BlockSpec((1, D), lambda i: (0, 0)),
            pl.BlockSpec((D, D), lambda i: (0, 0)),
            pl.BlockSpec((1, D), lambda i: (0, 0)),
        ],
        out_specs=pl.BlockSpec((BM, D), lambda i: (i, 0)),
        out_shape=jax.ShapeDtypeStruct((N, D), jnp.float32),
    )(x, W1, b1.reshape(1, D), W2, b2.reshape(1, D))


def _gmm(h, W, deg2):
    def body(h_ref, w_ref, da_ref, db_ref, o_ref):
        dinv = lax.rsqrt(da_ref[...] + db_ref[...] + 1.0)
        g = dinv * jnp.dot(h_ref[...], w_ref[...],
                           preferred_element_type=jnp.float32)
        o_ref[0] = g[:, :HD]
        o_ref[1] = g[:, HD:]

    return pl.pallas_call(
        body,
        grid=(NB,),
        in_specs=[
            pl.BlockSpec((BM, D), lambda i: (i, 0)),
            pl.BlockSpec((D, D), lambda i: (0, 0)),
            pl.BlockSpec((BM, 1), lambda i: (i, 0)),
            pl.BlockSpec((BM, 1), lambda i: (NB + i, 0)),
        ],
        out_specs=pl.BlockSpec((2, BM, HD), lambda i: (0, i, 0)),
        out_shape=jax.ShapeDtypeStruct((2, N, HD), jnp.float32),
    )(h, W, deg2, deg2)


def _post(acc2, g2, deg2, b):
    def body(aL, aR, gL, gR, da, db, b_ref, o_ref):
        dinv = lax.rsqrt(da[...] + db[...] + 1.0)
        o_ref[:, :HD] = jnp.maximum(
            dinv * (aL[...] + gL[...]) + b_ref[:, :HD], 0.0)
        o_ref[:, HD:] = jnp.maximum(
            dinv * (aR[...] + gR[...]) + b_ref[:, HD:], 0.0)

    return pl.pallas_call(
        body,
        grid=(NB,),
        in_specs=[
            pl.BlockSpec((BM, HD), lambda i: (i, 0)),
            pl.BlockSpec((BM, HD), lambda i: (NB + i, 0)),
            pl.BlockSpec((BM, HD), lambda i: (i, 0)),
            pl.BlockSpec((BM, HD), lambda i: (NB + i, 0)),
            pl.BlockSpec((BM, 1), lambda i: (i, 0)),
            pl.BlockSpec((BM, 1), lambda i: (NB + i, 0)),
            pl.BlockSpec((1, D), lambda i: (0, 0)),
        ],
        out_specs=pl.BlockSpec((BM, D), lambda i: (i, 0)),
        out_shape=jax.ShapeDtypeStruct((N, D), jnp.float32),
    )(acc2, acc2, g2, g2, deg2, deg2, b.reshape(1, D))


def _head(h, idx2, ea0, Wp1, bp1, Wp2, bp2, Wf1, bf1, Wf2, bf2,
          Ws1, bs1, Ws2, bs2):
    def body(idx_ref, hs_ref, hd_ref, ea_ref,
             wp1, bp1r, wp2, bp2r, wf1, bf1r, wf2, bf2r,
             ws1, bs1r, ws2, bs2r, c_ref, s_ref):
        rs = idx_ref[0] % 8
        rd = idx_ref[1] % 8
        hs = hs_ref[pl.ds(rs, 1), :]
        hd = hd_ref[pl.ds(rd, 1), :]
        dot = functools.partial(jnp.dot, preferred_element_type=jnp.float32)
        e = dot(hs, wp1[:D, :]) + dot(hd, wp1[D:, :]) + bp1r[...]
        e = jnp.maximum(e, 0.0)
        e = dot(e, wp2[...]) + bp2r[...]
        f = dot(e, wf1[:HD, :]) + dot(ea_ref[...], wf1[HD:, :]) + bf1r[...]
        f = jnp.maximum(f, 0.0)
        f = dot(f, wf2[...]) + bf2r[...]
        c_ref[...] = f
        s = jnp.maximum(dot(f, ws1[...]) + bs1r[...], 0.0)
        z = dot(s, ws2[...]) + bs2r[...]
        s_ref[...] = 1.0 / (1.0 + jnp.exp(-z))

    grid_spec = pltpu.PrefetchScalarGridSpec(
        num_scalar_prefetch=1,
        grid=(1,),
        in_specs=[
            pl.BlockSpec((8, D), lambda i, idx: (idx[0] // 8, 0)),
            pl.BlockSpec((8, D), lambda i, idx: (idx[1] // 8, 0)),
            pl.BlockSpec((1, 16), lambda i, idx: (0, 0)),
            pl.BlockSpec((2 * D, D), lambda i, idx: (0, 0)),
            pl.BlockSpec((1, D), lambda i, idx: (0, 0)),
            pl.BlockSpec((D, HD), lambda i, idx: (0, 0)),
            pl.BlockSpec((1, HD), lambda i, idx: (0, 0)),
            pl.BlockSpec((HD + 16, D), lambda i, idx: (0, 0)),
            pl.BlockSpec((1, D), lambda i, idx: (0, 0)),
            pl.BlockSpec((D, HD), lambda i, idx: (0, 0)),
            pl.BlockSpec((1, HD), lambda i, idx: (0, 0)),
            pl.BlockSpec((HD, D), lambda i, idx: (0, 0)),
            pl.BlockSpec((1, D), lambda i, idx: (0, 0)),
            pl.BlockSpec((D, 1), lambda i, idx: (0, 0)),
            pl.BlockSpec((1, 1), lambda i, idx: (0, 0)),
        ],
        out_specs=[
            pl.BlockSpec((1, HD), lambda i, idx: (0, 0)),
            pl.BlockSpec((1, 1), lambda i, idx: (0, 0)),
        ],
    )
    return pl.pallas_call(
        body,
        grid_spec=grid_spec,
        out_shape=[
            jax.ShapeDtypeStruct((1, HD), jnp.float32),
            jax.ShapeDtypeStruct((1, 1), jnp.float32),
        ],
    )(idx2, h, h, ea0,
      Wp1, bp1.reshape(1, D), Wp2, bp2.reshape(1, HD),
      Wf1, bf1.reshape(1, D), Wf2, bf2.reshape(1, HD),
      Ws1, bs1.reshape(1, D), Ws2, bs2.reshape(1, 1))


def kernel(x, edge_index, edge_attr,
           W_ne1, b_ne1, W_ne2, b_ne2,
           Wc1, bc1, Wc2, bc2, Wc3, bc3,
           Wp1, bp1, Wp2, bp2,
           Wf1, bf1, Wf2, bf2,
           Ws1, bs1, Ws2, bs2):
    ei = edge_index.astype(jnp.int32)
    src, dst = ei[0], ei[1]
    srcr = src.reshape(NSUB, CPT, K)
    src3 = jnp.concatenate([srcr, srcr + N]).reshape(2 * NSUB * NG, G, K)
    dst3 = dst.reshape(NSUB * NG, G, K)
    idx2 = ei[:, 0]
    ea0 = edge_attr[0:1]

    deg2 = _sc_degree(dst)[:, :1]
    h = _enc(x, W_ne1, b_ne1, W_ne2, b_ne2)
    for (W, b) in ((Wc1, bc1), (Wc2, bc2), (Wc3, bc3)):
        g2 = _gmm(h, W, deg2).reshape(2 * N, HD)
        acc2 = _sc_segsum(g2, src3, dst3)
        h = _post(acc2, g2, deg2, b)

    center, score = _head(h, idx2, ea0, Wp1, bp1, Wp2, bp2,
                          Wf1, bf1, Wf2, bf2, Ws1, bs1, Ws2, bs2)
    return center, score.reshape(1)
